# baseline (speedup 1.0000x reference)
import functools

import jax
import jax.numpy as jnp
from jax import lax
from jax.experimental import pallas as pl
from jax.experimental.pallas import tpu as pltpu
from jax.experimental.pallas import tpu_sc as plsc

_WIN = 128
_OVERLAP = 0


def kernel(x):
    B, T = x.shape
    stride = _WIN - _OVERLAP
    n_windows = T // _WIN

    nc = 1
    rows_per_core = B // nc

    mesh = plsc.ScalarSubcoreMesh(axis_name="c", num_cores=nc)

    @functools.partial(
        pl.kernel,
        mesh=mesh,
        out_type=jax.ShapeDtypeStruct((B * n_windows * _WIN,), x.dtype),
        scratch_types=[pltpu.SemaphoreType.DMA],
    )
    def _rolling_window(x_hbm, out_hbm, sem):
        cid = lax.axis_index("c")
        win_per_dma = n_windows // 4
        copies = []
        for j in range(rows_per_core):
            b = cid * rows_per_core + j
            for k in range(4):
                w0 = k * win_per_dma
                src = x_hbm.at[b, pl.ds(w0 * stride, win_per_dma * _WIN)]
                dst = out_hbm.at[
                    pl.ds((b * n_windows + w0) * _WIN, win_per_dma * _WIN)
                ]
                copies.append(pltpu.make_async_copy(src, dst, sem))
        for c in copies:
            c.start()
        pltpu.make_async_copy(out_hbm, out_hbm, sem).wait()

    out_flat = _rolling_window(x)
    return out_flat.reshape(B, n_windows, _WIN)
